# Initial kernel scaffold; baseline (speedup 1.0000x reference)
import jax
import jax.numpy as jnp
from jax.experimental import pallas as pl


def kernel(byte_ids, embed_weight, cls_weight, cls_bias):
    raise NotImplementedError("write your pallas kernel here")


# speedup vs baseline: 1.7355x; 1.7355x over previous
import functools

import jax
import jax.numpy as jnp
from jax import lax
from jax.experimental import pallas as pl
from jax.experimental.pallas import tpu as pltpu
from jax.experimental.pallas import tpu_sc as plsc

SEQ_LEN = 8192
VOCAB = 256
NUM_CORES = 2
NUM_SUBCORES = 16
NUM_WORKERS = NUM_CORES * NUM_SUBCORES
IDS_PER_WORKER = SEQ_LEN // NUM_WORKERS
LANES = 16

_mesh = plsc.VectorSubcoreMesh(
    core_axis_name="c", subcore_axis_name="s",
    num_cores=NUM_CORES, num_subcores=NUM_SUBCORES,
)


@functools.partial(
    pl.kernel,
    out_type=jax.ShapeDtypeStruct((NUM_WORKERS, VOCAB), jnp.float32),
    mesh=_mesh,
    scratch_types=[
        pltpu.VMEM((IDS_PER_WORKER,), jnp.int32),
        pltpu.VMEM((VOCAB,), jnp.float32),
    ],
    compiler_params=pltpu.CompilerParams(needs_layout_passes=False),
)
def _sc_histogram(ids_hbm, out_hbm, ids_v, hist_v):
    wid = lax.axis_index("s") * NUM_CORES + lax.axis_index("c")
    base = wid * IDS_PER_WORKER
    pltpu.sync_copy(ids_hbm.at[pl.ds(base, IDS_PER_WORKER)], ids_v)
    zeros = jnp.zeros((LANES,), jnp.float32)
    for i in range(VOCAB // LANES):
        hist_v[pl.ds(i * LANES, LANES)] = zeros
    ones = jnp.ones((LANES,), jnp.float32)
    for i in range(IDS_PER_WORKER // LANES):
        idx = ids_v[pl.ds(i * LANES, LANES)]
        plsc.addupdate_scatter(hist_v, [idx], ones)
    pltpu.sync_copy(hist_v, out_hbm.at[wid])


def _tc_dense(partials_ref, embed_ref, cls_ref, bias_ref, out_ref):
    counts = jnp.sum(partials_ref[...], axis=0, keepdims=True)
    mean = lax.dot_general(
        counts, embed_ref[...],
        (((1,), (0,)), ((), ())),
        preferred_element_type=jnp.float32,
    ) * (1.0 / SEQ_LEN)
    logits = lax.dot_general(
        mean, cls_ref[...],
        (((1,), (1,)), ((), ())),
        preferred_element_type=jnp.float32,
    ) + bias_ref[...]
    out_ref[...] = 1.0 / (1.0 + jnp.exp(-logits))


def kernel(byte_ids, embed_weight, cls_weight, cls_bias):
    ids = byte_ids.reshape(-1).astype(jnp.int32)
    partials = _sc_histogram(ids)
    scores = pl.pallas_call(
        _tc_dense,
        out_shape=jax.ShapeDtypeStruct((1, cls_weight.shape[0]), jnp.float32),
    )(partials, embed_weight, cls_weight, cls_bias.reshape(1, -1))
    return scores.reshape(-1)
